# baseline (speedup 1.0000x reference)
import jax
import jax.numpy as jnp
from jax.experimental import pallas as pl
from jax.experimental.pallas import tpu as pltpu

N_TOKENS = 771
DIM = 512
NUM_SPATIAL = 256
NBUF = 6


def _bias_body(te_ref, sp_ref, gl_ref, out_ref):
    te = te_ref[...]
    sp = sp_ref[0]
    gl = gl_ref[0]
    bias = jnp.concatenate(
        [
            te[0:1] + gl[0:1],
            te[1:2] + gl[1:2],
            te[2:3] + sp,
            te[3:4] + sp,
            te[4:5] + gl[2:3],
            te[5:6] + sp,
        ],
        axis=0,
    )
    out_ref[...] = bias


def _add_body(tok_hbm, bias_ref, out_hbm, in_buf, out_buf, in_sems, out_sems):
    B = tok_hbm.shape[0]

    def in_copy(i, slot):
        return pltpu.make_async_copy(tok_hbm.at[i], in_buf.at[slot], in_sems.at[slot])

    def out_copy(i, slot):
        return pltpu.make_async_copy(out_buf.at[slot], out_hbm.at[i], out_sems.at[slot])

    for s in range(NBUF):
        in_copy(s, s).start()

    bias = bias_ref[...]

    def step(i, _):
        slot = jax.lax.rem(i, NBUF)
        in_copy(i, slot).wait()

        @pl.when(i >= NBUF)
        def _():
            out_copy(i - NBUF, slot).wait()

        out_buf[slot] = in_buf[slot] + bias
        out_copy(i, slot).start()

        @pl.when(i + NBUF < B)
        def _():
            in_copy(i + NBUF, slot).start()

        return 0

    jax.lax.fori_loop(0, B, step, 0)

    def drain(i, _):
        out_copy(i, jax.lax.rem(i, NBUF)).wait()
        return 0

    jax.lax.fori_loop(B - NBUF, B, drain, 0)


def kernel(tokens, type_embedding, spatial_pos_embedding, global_pos_embedding):
    B, N, D = tokens.shape

    te = type_embedding
    sp = spatial_pos_embedding[0]
    gl = global_pos_embedding[0]
    bias = jnp.concatenate(
        [
            te[0:1] + gl[0:1],
            te[1:2] + gl[1:2],
            te[2:3] + sp,
            te[3:4] + sp,
            te[4:5] + gl[2:3],
            te[5:6] + sp,
        ],
        axis=0,
    )

    out = pl.pallas_call(
        _add_body,
        in_specs=[
            pl.BlockSpec(memory_space=pl.MemorySpace.ANY),
            pl.BlockSpec((N, D), lambda: (0, 0)),
        ],
        out_specs=pl.BlockSpec(memory_space=pl.MemorySpace.ANY),
        out_shape=jax.ShapeDtypeStruct((B, N, D), tokens.dtype),
        scratch_shapes=[
            pltpu.VMEM((NBUF, N, D), tokens.dtype),
            pltpu.VMEM((NBUF, N, D), tokens.dtype),
            pltpu.SemaphoreType.DMA((NBUF,)),
            pltpu.SemaphoreType.DMA((NBUF,)),
        ],
    )(tokens, bias)
    return out


# speedup vs baseline: 1.0344x; 1.0344x over previous
import jax
import jax.numpy as jnp
from jax.experimental import pallas as pl
from jax.experimental.pallas import tpu as pltpu

N_TOKENS = 771
DIM = 512
NUM_SPATIAL = 256
K = 4
NBUF = 2


def _bias_body(te_ref, sp_ref, gl_ref, out_ref):
    te = te_ref[...]
    sp = sp_ref[0]
    gl = gl_ref[0]
    bias = jnp.concatenate(
        [
            te[0:1] + gl[0:1],
            te[1:2] + gl[1:2],
            te[2:3] + sp,
            te[3:4] + sp,
            te[4:5] + gl[2:3],
            te[5:6] + sp,
        ],
        axis=0,
    )
    out_ref[...] = bias


def _add_body(tok_hbm, bias_ref, out_hbm, in_buf, out_buf, in_sems, out_sems):
    B = tok_hbm.shape[0]
    per = B // K

    def in_copy(k, j, slot):
        return pltpu.make_async_copy(
            tok_hbm.at[k * per + j], in_buf.at[k, slot], in_sems.at[k, slot]
        )

    def out_copy(k, j, slot):
        return pltpu.make_async_copy(
            out_buf.at[k, slot], out_hbm.at[k * per + j], out_sems.at[k, slot]
        )

    for k in range(K):
        for s in range(NBUF):
            in_copy(k, s, s).start()

    bias = bias_ref[...]

    def step(j, _):
        slot = jax.lax.rem(j, NBUF)
        for k in range(K):
            in_copy(k, j, slot).wait()

            @pl.when(j >= NBUF)
            def _(k=k):
                out_copy(k, j - NBUF, slot).wait()

            out_buf[k, slot] = in_buf[k, slot] + bias
            out_copy(k, j, slot).start()

            @pl.when(j + NBUF < per)
            def _(k=k):
                in_copy(k, j + NBUF, slot).start()

        return 0

    jax.lax.fori_loop(0, per, step, 0)

    def drain(j, _):
        slot = jax.lax.rem(j, NBUF)
        for k in range(K):
            out_copy(k, j, slot).wait()
        return 0

    jax.lax.fori_loop(per - NBUF, per, drain, 0)


def kernel(tokens, type_embedding, spatial_pos_embedding, global_pos_embedding):
    B, N, D = tokens.shape

    bias = pl.pallas_call(
        _bias_body,
        out_shape=jax.ShapeDtypeStruct((N, D), tokens.dtype),
    )(type_embedding, spatial_pos_embedding, global_pos_embedding)

    out = pl.pallas_call(
        _add_body,
        in_specs=[
            pl.BlockSpec(memory_space=pl.MemorySpace.ANY),
            pl.BlockSpec((N, D), lambda: (0, 0)),
        ],
        out_specs=pl.BlockSpec(memory_space=pl.MemorySpace.ANY),
        out_shape=jax.ShapeDtypeStruct((B, N, D), tokens.dtype),
        scratch_shapes=[
            pltpu.VMEM((K, NBUF, N, D), tokens.dtype),
            pltpu.VMEM((K, NBUF, N, D), tokens.dtype),
            pltpu.SemaphoreType.DMA((K, NBUF)),
            pltpu.SemaphoreType.DMA((K, NBUF)),
        ],
    )(tokens, bias)
    return out
